# Initial kernel scaffold; baseline (speedup 1.0000x reference)
import jax
import jax.numpy as jnp
from jax.experimental import pallas as pl
from jax.experimental.pallas import tpu as pltpu
from jax.experimental.pallas import tpu_sc as plsc

KV_HEADS = 4
HEAD_DIM = 128
GATE_DIM = 128
KV = KV_HEADS * HEAD_DIM

GATHER_WINDOW = 64
TC_BLOCK = 512


def _sc_gather(embed_table, flat_ids):
    n = flat_ids.shape[0]
    ids2d = flat_ids.reshape(1, n)
    mesh = plsc.VectorSubcoreMesh(core_axis_name="core",
                                  subcore_axis_name="subcore")

    @pl.kernel(
        out_type=jax.ShapeDtypeStruct((n, KV), embed_table.dtype),
        mesh=mesh,
    )
    def gather_kernel(table_hbm, ids_hbm, out_hbm):
        def body(ids_vmem, out_vmem):
            pltpu.sync_copy(table_hbm.at[ids_vmem.at[0]], out_vmem)

        pltpu.emit_pipeline(
            body,
            grid=(n // GATHER_WINDOW,),
            in_specs=[pl.BlockSpec((1, GATHER_WINDOW),
                                   index_map=lambda i: (0, i))],
            out_specs=[pl.BlockSpec((GATHER_WINDOW, KV),
                                    index_map=lambda i: (i, 0))],
            core_axis_name=("core", "subcore"),
            dimension_semantics=(pltpu.PARALLEL,),
        )(ids_hbm, out_hbm)

    return gather_kernel(embed_table, ids2d)


def _tc_scale_kernel(x_ref, ve_ref, w_ref, b_ref, out_ref):
    logits = jnp.dot(x_ref[...], w_ref[...],
                     preferred_element_type=jnp.float32) + b_ref[...]
    out_ref[...] = 2.0 * jax.nn.sigmoid(logits) * ve_ref[...]


def _tc_scale(x2d, ve, w_big, b_big):
    n, d = x2d.shape
    grid = (n // TC_BLOCK,)
    return pl.pallas_call(
        _tc_scale_kernel,
        out_shape=jax.ShapeDtypeStruct((n, KV), jnp.float32),
        grid=grid,
        in_specs=[
            pl.BlockSpec((TC_BLOCK, GATE_DIM), lambda i: (i, 0)),
            pl.BlockSpec((TC_BLOCK, KV), lambda i: (i, 0)),
            pl.BlockSpec((GATE_DIM, KV), lambda i: (0, 0)),
            pl.BlockSpec((1, KV), lambda i: (0, 0)),
        ],
        out_specs=pl.BlockSpec((TC_BLOCK, KV), lambda i: (i, 0)),
    )(x2d, ve, w_big, b_big)


def kernel(input_ids, x, layer_idx, embed_table, gate_W, gate_b):
    b, t = input_ids.shape
    n = b * t
    d = x.shape[-1]

    w_big = jnp.broadcast_to(gate_W.T[:, :, None],
                             (GATE_DIM, KV_HEADS, HEAD_DIM)).reshape(GATE_DIM, KV)
    b_big = jnp.broadcast_to(gate_b[:, None],
                             (KV_HEADS, HEAD_DIM)).reshape(1, KV)

    flat_ids = input_ids.reshape(n)
    ve = _sc_gather(embed_table, flat_ids)

    x2d = x.reshape(n, d)
    out = _tc_scale(x2d, ve, w_big, b_big)
    return out.reshape(b, t, KV_HEADS, HEAD_DIM)


# speedup vs baseline: 1.0434x; 1.0434x over previous
import functools

import jax
import jax.numpy as jnp
from jax import lax
from jax.experimental import pallas as pl
from jax.experimental.pallas import tpu as pltpu
from jax.experimental.pallas import tpu_sc as plsc

KV_HEADS = 4
HEAD_DIM = 128
GATE_DIM = 128
KV = KV_HEADS * HEAD_DIM

NUM_WORKERS = 32
GATHER_WINDOW = 64
TC_BLOCK = 512


def _sc_gather(embed_table, flat_ids):
    n = flat_ids.shape[0]
    per_w = n // NUM_WORKERS
    n_win = per_w // GATHER_WINDOW
    mesh = plsc.VectorSubcoreMesh(core_axis_name="c", subcore_axis_name="s")

    @functools.partial(
        pl.kernel,
        out_type=jax.ShapeDtypeStruct((n, KV), embed_table.dtype),
        mesh=mesh,
        scratch_types=[
            pltpu.VMEM((per_w,), jnp.int32),
            pltpu.VMEM((GATHER_WINDOW, KV), jnp.float32),
            pltpu.VMEM((GATHER_WINDOW, KV), jnp.float32),
            pltpu.SemaphoreType.DMA,
            pltpu.SemaphoreType.DMA,
            pltpu.SemaphoreType.DMA,
            pltpu.SemaphoreType.DMA,
        ],
    )
    def gather_kernel(table_hbm, ids_hbm, out_hbm,
                      idx_v, buf0, buf1, gs0, gs1, ws0, ws1):
        wid = lax.axis_index("s") * 2 + lax.axis_index("c")
        base = wid * per_w
        pltpu.sync_copy(ids_hbm.at[pl.ds(base, per_w)], idx_v)

        bufs = (buf0, buf1)
        gsems = (gs0, gs1)
        wsems = (ws0, ws1)
        gathers = [None, None]
        writes = [None, None]
        gathers[0] = pltpu.async_copy(
            table_hbm.at[idx_v.at[pl.ds(0, GATHER_WINDOW)]], bufs[0], gsems[0])
        for w in range(n_win):
            b = w % 2
            gathers[b].wait()
            if w + 1 < n_win:
                b2 = (w + 1) % 2
                if writes[b2] is not None:
                    writes[b2].wait()
                gathers[b2] = pltpu.async_copy(
                    table_hbm.at[idx_v.at[pl.ds((w + 1) * GATHER_WINDOW,
                                                GATHER_WINDOW)]],
                    bufs[b2], gsems[b2])
            writes[b] = pltpu.async_copy(
                bufs[b],
                out_hbm.at[pl.ds(base + w * GATHER_WINDOW, GATHER_WINDOW)],
                wsems[b])
        for wr in writes:
            if wr is not None:
                wr.wait()

    return gather_kernel(embed_table, flat_ids)


def _tc_scale_kernel(x_ref, ve_ref, w_ref, b_ref, out_ref):
    logits = jnp.dot(x_ref[...], w_ref[...],
                     preferred_element_type=jnp.float32) + b_ref[...]
    out_ref[...] = 2.0 * jax.nn.sigmoid(logits) * ve_ref[...]


def _tc_scale(x2d, ve, w_big, b_big):
    n, d = x2d.shape
    grid = (n // TC_BLOCK,)
    return pl.pallas_call(
        _tc_scale_kernel,
        out_shape=jax.ShapeDtypeStruct((n, KV), jnp.float32),
        grid=grid,
        in_specs=[
            pl.BlockSpec((TC_BLOCK, GATE_DIM), lambda i: (i, 0)),
            pl.BlockSpec((TC_BLOCK, KV), lambda i: (i, 0)),
            pl.BlockSpec((GATE_DIM, KV), lambda i: (0, 0)),
            pl.BlockSpec((1, KV), lambda i: (0, 0)),
        ],
        out_specs=pl.BlockSpec((TC_BLOCK, KV), lambda i: (i, 0)),
    )(x2d, ve, w_big, b_big)


def kernel(input_ids, x, layer_idx, embed_table, gate_W, gate_b):
    b, t = input_ids.shape
    n = b * t
    d = x.shape[-1]

    w_big = jnp.broadcast_to(gate_W.T[:, :, None],
                             (GATE_DIM, KV_HEADS, HEAD_DIM)).reshape(GATE_DIM, KV)
    b_big = jnp.broadcast_to(gate_b[:, None],
                             (KV_HEADS, HEAD_DIM)).reshape(1, KV)

    flat_ids = input_ids.reshape(n)
    ve = _sc_gather(embed_table, flat_ids)

    x2d = x.reshape(n, d)
    out = _tc_scale(x2d, ve, w_big, b_big)
    return out.reshape(b, t, KV_HEADS, HEAD_DIM)


# speedup vs baseline: 1.3703x; 1.3133x over previous
import functools

import jax
import jax.numpy as jnp
from jax import lax
from jax.experimental import pallas as pl
from jax.experimental.pallas import tpu as pltpu
from jax.experimental.pallas import tpu_sc as plsc

KV_HEADS = 4
HEAD_DIM = 128
GATE_DIM = 128
KV = KV_HEADS * HEAD_DIM

NUM_WORKERS = 32
GATHER_WINDOW = 64
TC_BLOCK = 512


def _sc_gather(embed_table, flat_ids):
    n = flat_ids.shape[0]
    per_w = n // NUM_WORKERS
    n_win = per_w // GATHER_WINDOW
    mesh = plsc.VectorSubcoreMesh(core_axis_name="c", subcore_axis_name="s")

    @functools.partial(
        pl.kernel,
        out_type=jax.ShapeDtypeStruct((n, KV), embed_table.dtype),
        mesh=mesh,
        scratch_types=[
            pltpu.VMEM((per_w,), jnp.int32),
            pltpu.VMEM((GATHER_WINDOW, KV), jnp.float32),
            pltpu.VMEM((GATHER_WINDOW, KV), jnp.float32),
            pltpu.SemaphoreType.DMA,
            pltpu.SemaphoreType.DMA,
            pltpu.SemaphoreType.DMA,
            pltpu.SemaphoreType.DMA,
        ],
    )
    def gather_kernel(table_hbm, ids_hbm, out_hbm,
                      idx_v, buf0, buf1, gs0, gs1, ws0, ws1):
        wid = lax.axis_index("s") * 2 + lax.axis_index("c")
        base = wid * per_w
        pltpu.sync_copy(ids_hbm.at[pl.ds(base, per_w)], idx_v)

        bufs = (buf0, buf1)
        gsems = (gs0, gs1)
        wsems = (ws0, ws1)
        gathers = [None, None]
        writes = [None, None]
        gathers[0] = pltpu.async_copy(
            table_hbm.at[idx_v.at[pl.ds(0, GATHER_WINDOW)]], bufs[0], gsems[0])
        for w in range(n_win):
            b = w % 2
            gathers[b].wait()
            if w + 1 < n_win:
                b2 = (w + 1) % 2
                if writes[b2] is not None:
                    writes[b2].wait()
                gathers[b2] = pltpu.async_copy(
                    table_hbm.at[idx_v.at[pl.ds((w + 1) * GATHER_WINDOW,
                                                GATHER_WINDOW)]],
                    bufs[b2], gsems[b2])
            writes[b] = pltpu.async_copy(
                bufs[b],
                out_hbm.at[pl.ds(base + w * GATHER_WINDOW, GATHER_WINDOW)],
                wsems[b])
        for wr in writes:
            if wr is not None:
                wr.wait()

    return gather_kernel(embed_table, flat_ids)


def _tc_scale_kernel(x_ref, ve_ref, w_ref, b_ref, out_ref):
    xg = x_ref[0]
    logits = jnp.dot(xg, w_ref[...],
                     preferred_element_type=jnp.float32) + b_ref[...]
    s = 2.0 * jax.nn.sigmoid(logits)
    for h in range(KV_HEADS):
        out_ref[0, :, h, :] = (ve_ref[:, h * HEAD_DIM:(h + 1) * HEAD_DIM]
                               * s[:, h][:, None])


def _tc_scale(x, ve, w_t, b_row, bsz, seq):
    t_blocks = seq // TC_BLOCK
    return pl.pallas_call(
        _tc_scale_kernel,
        out_shape=jax.ShapeDtypeStruct((bsz, seq, KV_HEADS, HEAD_DIM),
                                       jnp.float32),
        grid=(bsz, t_blocks),
        in_specs=[
            pl.BlockSpec((1, TC_BLOCK, GATE_DIM), lambda bi, ti: (bi, ti, 0)),
            pl.BlockSpec((TC_BLOCK, KV),
                         lambda bi, ti: (bi * t_blocks + ti, 0)),
            pl.BlockSpec((GATE_DIM, KV_HEADS), lambda bi, ti: (0, 0)),
            pl.BlockSpec((1, KV_HEADS), lambda bi, ti: (0, 0)),
        ],
        out_specs=pl.BlockSpec((1, TC_BLOCK, KV_HEADS, HEAD_DIM),
                               lambda bi, ti: (bi, ti, 0, 0)),
    )(x, ve, w_t, b_row)


def kernel(input_ids, x, layer_idx, embed_table, gate_W, gate_b):
    b, t = input_ids.shape
    n = b * t

    flat_ids = input_ids.reshape(n)
    ve = _sc_gather(embed_table, flat_ids)

    return _tc_scale(x, ve, gate_W.T, gate_b.reshape(1, KV_HEADS), b, t)


# speedup vs baseline: 1.3712x; 1.0007x over previous
import functools

import jax
import jax.numpy as jnp
from jax import lax
from jax.experimental import pallas as pl
from jax.experimental.pallas import tpu as pltpu
from jax.experimental.pallas import tpu_sc as plsc

KV_HEADS = 4
HEAD_DIM = 128
GATE_DIM = 128
KV = KV_HEADS * HEAD_DIM

NUM_WORKERS = 32
GATHER_WINDOW = 64
TC_BLOCK = 512


def _sc_gather(embed_table, ids2d):
    n = ids2d.shape[0] * ids2d.shape[1]
    per_w = n // NUM_WORKERS
    n_win = per_w // GATHER_WINDOW
    mesh = plsc.VectorSubcoreMesh(core_axis_name="c", subcore_axis_name="s")

    @functools.partial(
        pl.kernel,
        out_type=jax.ShapeDtypeStruct((n, KV), embed_table.dtype),
        mesh=mesh,
        scratch_types=[
            pltpu.VMEM((per_w,), jnp.int32),
            pltpu.VMEM((GATHER_WINDOW, KV), jnp.float32),
            pltpu.VMEM((GATHER_WINDOW, KV), jnp.float32),
            pltpu.SemaphoreType.DMA,
            pltpu.SemaphoreType.DMA,
            pltpu.SemaphoreType.DMA,
            pltpu.SemaphoreType.DMA,
        ],
    )
    def gather_kernel(table_hbm, ids_hbm, out_hbm,
                      idx_v, buf0, buf1, gs0, gs1, ws0, ws1):
        wid = lax.axis_index("s") * 2 + lax.axis_index("c")
        base = wid * per_w
        w_per_row = ids_hbm.shape[1] // per_w
        pltpu.sync_copy(
            ids_hbm.at[wid // w_per_row,
                       pl.ds((wid % w_per_row) * per_w, per_w)],
            idx_v)

        bufs = (buf0, buf1)
        gsems = (gs0, gs1)
        wsems = (ws0, ws1)
        gathers = [None, None]
        writes = [None, None]
        gathers[0] = pltpu.async_copy(
            table_hbm.at[idx_v.at[pl.ds(0, GATHER_WINDOW)]], bufs[0], gsems[0])
        for w in range(n_win):
            b = w % 2
            gathers[b].wait()
            if w + 1 < n_win:
                b2 = (w + 1) % 2
                if writes[b2] is not None:
                    writes[b2].wait()
                gathers[b2] = pltpu.async_copy(
                    table_hbm.at[idx_v.at[pl.ds((w + 1) * GATHER_WINDOW,
                                                GATHER_WINDOW)]],
                    bufs[b2], gsems[b2])
            writes[b] = pltpu.async_copy(
                bufs[b],
                out_hbm.at[pl.ds(base + w * GATHER_WINDOW, GATHER_WINDOW)],
                wsems[b])
        for wr in writes:
            if wr is not None:
                wr.wait()

    return gather_kernel(embed_table, ids2d)


GATE_BLOCK = 1024


def _tc_gate_kernel(x_ref, w_ref, b_ref, s_ref):
    logits = jnp.dot(x_ref[0], w_ref[...],
                     preferred_element_type=jnp.float32) + b_ref[...]
    s_ref[...] = 2.0 * jax.nn.sigmoid(logits)


def _tc_gate(x, w_t, b_row):
    bsz, seq, _ = x.shape
    t_blocks = seq // GATE_BLOCK
    return pl.pallas_call(
        _tc_gate_kernel,
        out_shape=jax.ShapeDtypeStruct((bsz * seq, KV_HEADS), jnp.float32),
        grid=(bsz, t_blocks),
        in_specs=[
            pl.BlockSpec((1, GATE_BLOCK, GATE_DIM), lambda bi, ti: (bi, ti, 0)),
            pl.BlockSpec((GATE_DIM, KV_HEADS), lambda bi, ti: (0, 0)),
            pl.BlockSpec((1, KV_HEADS), lambda bi, ti: (0, 0)),
        ],
        out_specs=pl.BlockSpec((GATE_BLOCK, KV_HEADS),
                               lambda bi, ti: (bi * t_blocks + ti, 0)),
    )(x, w_t, b_row)


def _tc_scale_kernel(ve_ref, s_ref, out_ref):
    s = s_ref[...]
    for h in range(KV_HEADS):
        out_ref[0, :, h, :] = (ve_ref[:, h * HEAD_DIM:(h + 1) * HEAD_DIM]
                               * s[:, h][:, None])


def _tc_scale(ve, s, bsz, seq):
    t_blocks = seq // TC_BLOCK
    return pl.pallas_call(
        _tc_scale_kernel,
        out_shape=jax.ShapeDtypeStruct((bsz, seq, KV_HEADS, HEAD_DIM),
                                       jnp.float32),
        grid=(bsz, t_blocks),
        in_specs=[
            pl.BlockSpec((TC_BLOCK, KV),
                         lambda bi, ti: (bi * t_blocks + ti, 0)),
            pl.BlockSpec((TC_BLOCK, KV_HEADS),
                         lambda bi, ti: (bi * t_blocks + ti, 0)),
        ],
        out_specs=pl.BlockSpec((1, TC_BLOCK, KV_HEADS, HEAD_DIM),
                               lambda bi, ti: (bi, ti, 0, 0)),
    )(ve, s)


def kernel(input_ids, x, layer_idx, embed_table, gate_W, gate_b):
    b, t = input_ids.shape

    ve = _sc_gather(embed_table, input_ids)
    s = _tc_gate(x, gate_W.T, gate_b.reshape(1, KV_HEADS))
    return _tc_scale(ve, s, b, t)


# speedup vs baseline: 1.4946x; 1.0900x over previous
import functools

import jax
import jax.numpy as jnp
from jax import lax
from jax.experimental import pallas as pl
from jax.experimental.pallas import tpu as pltpu
from jax.experimental.pallas import tpu_sc as plsc

KV_HEADS = 4
HEAD_DIM = 128
GATE_DIM = 128
KV = KV_HEADS * HEAD_DIM

NUM_WORKERS = 32
GATHER_WINDOW = 64
TC_BLOCK = 1024


def _sc_gather(embed_table, ids2d):
    n = ids2d.shape[0] * ids2d.shape[1]
    per_w = n // NUM_WORKERS
    n_win = per_w // GATHER_WINDOW
    mesh = plsc.VectorSubcoreMesh(core_axis_name="c", subcore_axis_name="s")

    @functools.partial(
        pl.kernel,
        out_type=jax.ShapeDtypeStruct((n, KV), embed_table.dtype),
        mesh=mesh,
        scratch_types=[
            pltpu.VMEM((per_w,), jnp.int32),
            pltpu.VMEM((GATHER_WINDOW, KV), jnp.float32),
            pltpu.VMEM((GATHER_WINDOW, KV), jnp.float32),
            pltpu.SemaphoreType.DMA,
            pltpu.SemaphoreType.DMA,
            pltpu.SemaphoreType.DMA,
            pltpu.SemaphoreType.DMA,
        ],
    )
    def gather_kernel(table_hbm, ids_hbm, out_hbm,
                      idx_v, buf0, buf1, gs0, gs1, ws0, ws1):
        wid = lax.axis_index("s") * 2 + lax.axis_index("c")
        base = wid * per_w
        w_per_row = ids_hbm.shape[1] // per_w
        pltpu.sync_copy(
            ids_hbm.at[wid // w_per_row,
                       pl.ds((wid % w_per_row) * per_w, per_w)],
            idx_v)

        bufs = (buf0, buf1)
        gsems = (gs0, gs1)
        wsems = (ws0, ws1)
        gathers = [None, None]
        writes = [None, None]
        gathers[0] = pltpu.async_copy(
            table_hbm.at[idx_v.at[pl.ds(0, GATHER_WINDOW)]], bufs[0], gsems[0])
        for w in range(n_win):
            b = w % 2
            gathers[b].wait()
            if w + 1 < n_win:
                b2 = (w + 1) % 2
                if writes[b2] is not None:
                    writes[b2].wait()
                gathers[b2] = pltpu.async_copy(
                    table_hbm.at[idx_v.at[pl.ds((w + 1) * GATHER_WINDOW,
                                                GATHER_WINDOW)]],
                    bufs[b2], gsems[b2])
            writes[b] = pltpu.async_copy(
                bufs[b],
                out_hbm.at[pl.ds(base + w * GATHER_WINDOW, GATHER_WINDOW)],
                wsems[b])
        for wr in writes:
            if wr is not None:
                wr.wait()

    return gather_kernel(embed_table, ids2d)


GATE_BLOCK = 1024


def _tc_gate_kernel(x_ref, w_ref, b_ref, s_ref):
    logits = jnp.dot(x_ref[0], w_ref[...],
                     preferred_element_type=jnp.float32) + b_ref[...]
    s_ref[...] = 2.0 * jax.nn.sigmoid(logits)


def _tc_gate(x, w_t, b_row):
    bsz, seq, _ = x.shape
    t_blocks = seq // GATE_BLOCK
    return pl.pallas_call(
        _tc_gate_kernel,
        out_shape=jax.ShapeDtypeStruct((bsz * seq, KV_HEADS), jnp.float32),
        grid=(bsz, t_blocks),
        in_specs=[
            pl.BlockSpec((1, GATE_BLOCK, GATE_DIM), lambda bi, ti: (bi, ti, 0)),
            pl.BlockSpec((GATE_DIM, KV_HEADS), lambda bi, ti: (0, 0)),
            pl.BlockSpec((1, KV_HEADS), lambda bi, ti: (0, 0)),
        ],
        out_specs=pl.BlockSpec((GATE_BLOCK, KV_HEADS),
                               lambda bi, ti: (bi * t_blocks + ti, 0)),
    )(x, w_t, b_row)


def _tc_scale_kernel(ve_ref, s_ref, out_ref):
    s = s_ref[...]
    for h in range(KV_HEADS):
        out_ref[0, :, h, :] = (ve_ref[:, h * HEAD_DIM:(h + 1) * HEAD_DIM]
                               * s[:, h][:, None])


def _tc_scale(ve, s, bsz, seq):
    t_blocks = seq // TC_BLOCK
    return pl.pallas_call(
        _tc_scale_kernel,
        out_shape=jax.ShapeDtypeStruct((bsz, seq, KV_HEADS, HEAD_DIM),
                                       jnp.float32),
        grid=(bsz, t_blocks),
        in_specs=[
            pl.BlockSpec((TC_BLOCK, KV),
                         lambda bi, ti: (bi * t_blocks + ti, 0)),
            pl.BlockSpec((TC_BLOCK, KV_HEADS),
                         lambda bi, ti: (bi * t_blocks + ti, 0)),
        ],
        out_specs=pl.BlockSpec((1, TC_BLOCK, KV_HEADS, HEAD_DIM),
                               lambda bi, ti: (bi, ti, 0, 0)),
    )(ve, s)


def kernel(input_ids, x, layer_idx, embed_table, gate_W, gate_b):
    b, t = input_ids.shape

    ve = _sc_gather(embed_table, input_ids)
    s = _tc_gate(x, gate_W.T, gate_b.reshape(1, KV_HEADS))
    return _tc_scale(ve, s, b, t)


# speedup vs baseline: 1.5247x; 1.0202x over previous
import functools

import jax
import jax.numpy as jnp
from jax import lax
from jax.experimental import pallas as pl
from jax.experimental.pallas import tpu as pltpu
from jax.experimental.pallas import tpu_sc as plsc

KV_HEADS = 4
HEAD_DIM = 128
GATE_DIM = 128
KV = KV_HEADS * HEAD_DIM

NUM_WORKERS = 32
GATHER_WINDOW = 64
TC_BLOCK = 2048


def _sc_gather(embed_table, ids2d):
    n = ids2d.shape[0] * ids2d.shape[1]
    per_w = n // NUM_WORKERS
    n_win = per_w // GATHER_WINDOW
    mesh = plsc.VectorSubcoreMesh(core_axis_name="c", subcore_axis_name="s")

    @functools.partial(
        pl.kernel,
        out_type=jax.ShapeDtypeStruct((n, KV), embed_table.dtype),
        mesh=mesh,
        scratch_types=[
            pltpu.VMEM((per_w,), jnp.int32),
            pltpu.VMEM((GATHER_WINDOW, KV), jnp.float32),
            pltpu.VMEM((GATHER_WINDOW, KV), jnp.float32),
            pltpu.SemaphoreType.DMA,
            pltpu.SemaphoreType.DMA,
            pltpu.SemaphoreType.DMA,
            pltpu.SemaphoreType.DMA,
        ],
    )
    def gather_kernel(table_hbm, ids_hbm, out_hbm,
                      idx_v, buf0, buf1, gs0, gs1, ws0, ws1):
        wid = lax.axis_index("s") * 2 + lax.axis_index("c")
        base = wid * per_w
        w_per_row = ids_hbm.shape[1] // per_w
        pltpu.sync_copy(
            ids_hbm.at[wid // w_per_row,
                       pl.ds((wid % w_per_row) * per_w, per_w)],
            idx_v)

        bufs = (buf0, buf1)
        gsems = (gs0, gs1)
        wsems = (ws0, ws1)
        gathers = [None, None]
        writes = [None, None]
        gathers[0] = pltpu.async_copy(
            table_hbm.at[idx_v.at[pl.ds(0, GATHER_WINDOW)]], bufs[0], gsems[0])
        for w in range(n_win):
            b = w % 2
            gathers[b].wait()
            if w + 1 < n_win:
                b2 = (w + 1) % 2
                if writes[b2] is not None:
                    writes[b2].wait()
                gathers[b2] = pltpu.async_copy(
                    table_hbm.at[idx_v.at[pl.ds((w + 1) * GATHER_WINDOW,
                                                GATHER_WINDOW)]],
                    bufs[b2], gsems[b2])
            writes[b] = pltpu.async_copy(
                bufs[b],
                out_hbm.at[pl.ds(base + w * GATHER_WINDOW, GATHER_WINDOW)],
                wsems[b])
        for wr in writes:
            if wr is not None:
                wr.wait()

    return gather_kernel(embed_table, ids2d)


GATE_BLOCK = 1024


def _tc_gate_kernel(x_ref, w_ref, b_ref, s_ref):
    logits = jnp.dot(x_ref[0], w_ref[...],
                     preferred_element_type=jnp.float32) + b_ref[...]
    s_ref[...] = 2.0 * jax.nn.sigmoid(logits)


def _tc_gate(x, w_t, b_row):
    bsz, seq, _ = x.shape
    t_blocks = seq // GATE_BLOCK
    return pl.pallas_call(
        _tc_gate_kernel,
        out_shape=jax.ShapeDtypeStruct((bsz * seq, KV_HEADS), jnp.float32),
        grid=(bsz, t_blocks),
        in_specs=[
            pl.BlockSpec((1, GATE_BLOCK, GATE_DIM), lambda bi, ti: (bi, ti, 0)),
            pl.BlockSpec((GATE_DIM, KV_HEADS), lambda bi, ti: (0, 0)),
            pl.BlockSpec((1, KV_HEADS), lambda bi, ti: (0, 0)),
        ],
        out_specs=pl.BlockSpec((GATE_BLOCK, KV_HEADS),
                               lambda bi, ti: (bi * t_blocks + ti, 0)),
    )(x, w_t, b_row)


def _tc_scale_kernel(ve_ref, s_ref, out_ref):
    s = s_ref[...]
    for h in range(KV_HEADS):
        out_ref[0, :, h, :] = (ve_ref[:, h * HEAD_DIM:(h + 1) * HEAD_DIM]
                               * s[:, h][:, None])


def _tc_scale(ve, s, bsz, seq):
    t_blocks = seq // TC_BLOCK
    return pl.pallas_call(
        _tc_scale_kernel,
        out_shape=jax.ShapeDtypeStruct((bsz, seq, KV_HEADS, HEAD_DIM),
                                       jnp.float32),
        grid=(bsz, t_blocks),
        in_specs=[
            pl.BlockSpec((TC_BLOCK, KV),
                         lambda bi, ti: (bi * t_blocks + ti, 0)),
            pl.BlockSpec((TC_BLOCK, KV_HEADS),
                         lambda bi, ti: (bi * t_blocks + ti, 0)),
        ],
        out_specs=pl.BlockSpec((1, TC_BLOCK, KV_HEADS, HEAD_DIM),
                               lambda bi, ti: (bi, ti, 0, 0)),
    )(ve, s)


def kernel(input_ids, x, layer_idx, embed_table, gate_W, gate_b):
    b, t = input_ids.shape

    ve = _sc_gather(embed_table, input_ids)
    s = _tc_gate(x, gate_W.T, gate_b.reshape(1, KV_HEADS))
    return _tc_scale(ve, s, b, t)
